# baseline (speedup 1.0000x reference)
import jax
import jax.numpy as jnp
from jax.experimental import pallas as pl
from jax.experimental.pallas import tpu as pltpu
from jax.experimental.pallas import tpu_sc as plsc

B, L, U, F = 8, 64, 32, 2
VOCAB, EMB = 30522, 768
SPK_V, SPK_E = 16, 32
UT = 256
NEM = 7

_SC_NC, _SC_NS = 2, 16
_SC_NW = _SC_NC * _SC_NS
_N_IDX = F * B * L
_ROWS_PER_W = _N_IDX // _SC_NW


def _sc_gather_body(table_hbm, idx_hbm, out_hbm, idx_v, rows_v, sem):
    wid = jax.lax.axis_index("s") * _SC_NC + jax.lax.axis_index("c")
    base = wid * _ROWS_PER_W
    pltpu.sync_copy(idx_hbm.at[pl.ds(base, _ROWS_PER_W)], idx_v)
    pltpu.async_copy(table_hbm.at[idx_v], rows_v, sem).wait()
    pltpu.sync_copy(rows_v, out_hbm.at[pl.ds(base, _ROWS_PER_W)])


def _sc_gather(table, idx):
    return pl.kernel(
        _sc_gather_body,
        out_type=jax.ShapeDtypeStruct((_N_IDX, EMB), jnp.float32),
        mesh=plsc.VectorSubcoreMesh(core_axis_name="c", subcore_axis_name="s"),
        scratch_types=[
            pltpu.VMEM((_ROWS_PER_W,), jnp.int32),
            pltpu.VMEM((_ROWS_PER_W, EMB), jnp.float32),
            pltpu.SemaphoreType.DMA,
        ],
    )(table, idx)


def _tc_body(rows_ref, spk_ref, pairm_ref, g_ref, spkt_ref,
             wucw_ref, wucs_ref, buc_ref, wuew_ref, wues_ref, bue_ref,
             wecw_ref, wecs_ref, bec_ref, weew_ref, wees_ref, bee_ref,
             wut_ref, wem_ref, wv_ref, bv_ref, wo_ref, bo_ref,
             wspr_ref, bspf_ref,
             sut_ref, sem_ref, sspan_ref):
    f32 = jnp.float32
    e0 = (rows_ref[0, 0] + rows_ref[1, 0]) * 0.5
    oh = (spk_ref[0]
          == jax.lax.broadcasted_iota(jnp.int32, (L, SPK_V), 1)).astype(f32)
    spk = jnp.dot(oh, spkt_ref[...], preferred_element_type=f32)

    def ffn(ww, ws, bb):
        h = (jnp.dot(e0, ww[...], preferred_element_type=f32)
             + jnp.dot(spk, ws[...], preferred_element_type=f32)
             + bb[...])
        return jnp.where(h >= 0, h, 0.1 * h)

    utc = ffn(wucw_ref, wucs_ref, buc_ref)
    ute = ffn(wuew_ref, wues_ref, bue_ref)
    emc = ffn(wecw_ref, wecs_ref, bec_ref)
    eme = ffn(weew_ref, wees_ref, bee_ref)

    ones1 = jnp.ones((L, 1), f32)

    def aug(x):
        return jnp.concatenate([x, ones1], axis=1)

    xc_ut, ye_ut = aug(utc), aug(ute)
    xc_em, ye_em = aug(emc), aug(eme)
    for o in range(2):
        xw = jnp.dot(xc_ut, wut_ref[o], preferred_element_type=f32)
        sut_ref[0, o] = jax.lax.dot_general(
            xw, ye_ut, (((1,), (1,)), ((), ())), preferred_element_type=f32)
    for o in range(NEM):
        xw = jnp.dot(xc_em, wem_ref[o], preferred_element_type=f32)
        sem_ref[0, o] = jax.lax.dot_general(
            xw, ye_em, (((1,), (1,)), ((), ())), preferred_element_type=f32)

    vp = jnp.dot(eme, wv_ref[...], preferred_element_type=f32) + bv_ref[...]
    sc = jnp.dot(vp, wo_ref[...], preferred_element_type=f32) + bo_ref[...]
    spw = jax.lax.dot_general(
        sc, wspr_ref[...], (((1,), (1,)), ((), ())),
        preferred_element_type=f32) + bspf_ref[...]
    spw = jnp.where(spw >= 0, spw, 0.1 * spw)
    m2 = (g_ref[0] != 0) & (pairm_ref[0] != 0)
    sspan_ref[0] = jnp.where(m2, spw, jnp.float32(-1.0))


def kernel(words, speakers, pad_mask, graphs, word_table, spk_table,
           Wuc, buc, Wue, bue, Wec, bec, Wee, bee, W_ut, W_em,
           Wq, bq, Wk, bk, Wv, bv, Wo, bo, Wsp, bsp):
    f32 = jnp.float32
    idx = jnp.transpose(words[:, :, 0, :], (2, 0, 1)).reshape(_N_IDX)
    idx = idx.astype(jnp.int32)
    rows = _sc_gather(word_table, idx)
    rows4 = rows.reshape(F, B, L, EMB)

    spk_i = jnp.broadcast_to(speakers[:, :, None], (B, L, SPK_V)).astype(jnp.int32)
    pair_i = (pad_mask[:, :, None] & pad_mask[:, None, :]).astype(jnp.int32)
    graphs_i = graphs.astype(jnp.int32)
    b2 = lambda v: v.reshape(1, -1).astype(f32)

    const2 = lambda b: (0, 0)
    const3 = lambda b: (0, 0, 0)
    w_spec = pl.BlockSpec((EMB, UT), const2)
    s_spec = pl.BlockSpec((SPK_E, UT), lambda b: (24, 0))
    b_spec = pl.BlockSpec((1, UT), const2)
    in_specs = [
        pl.BlockSpec((F, 1, L, EMB), lambda b: (0, b, 0, 0)),
        pl.BlockSpec((1, L, SPK_V), lambda b: (b, 0, 0)),
        pl.BlockSpec((1, L, L), lambda b: (b, 0, 0)),
        pl.BlockSpec((1, L, L), lambda b: (b, 0, 0)),
        pl.BlockSpec((SPK_V, SPK_E), const2),
        w_spec, s_spec, b_spec,
        w_spec, s_spec, b_spec,
        w_spec, s_spec, b_spec,
        w_spec, s_spec, b_spec,
        pl.BlockSpec((2, UT + 1, UT + 1), const3),
        pl.BlockSpec((NEM, UT + 1, UT + 1), const3),
        pl.BlockSpec((UT, EMB), const2),
        pl.BlockSpec((1, EMB), const2),
        pl.BlockSpec((EMB, EMB), const2),
        pl.BlockSpec((1, EMB), const2),
        pl.BlockSpec((L, EMB), const2),
        pl.BlockSpec((L, L), const2),
    ]
    out_specs = (
        pl.BlockSpec((1, 2, L, L), lambda b: (b, 0, 0, 0)),
        pl.BlockSpec((1, NEM, L, L), lambda b: (b, 0, 0, 0)),
        pl.BlockSpec((1, L, L), lambda b: (b, 0, 0)),
    )
    sut_k, sem_k, span_k = pl.pallas_call(
        _tc_body,
        grid=(B,),
        in_specs=in_specs,
        out_specs=out_specs,
        out_shape=(
            jax.ShapeDtypeStruct((B, 2, L, L), f32),
            jax.ShapeDtypeStruct((B, NEM, L, L), f32),
            jax.ShapeDtypeStruct((B, L, L), f32),
        ),
    )(rows4, spk_i, pair_i, graphs_i, spk_table,
      Wuc, Wuc, b2(buc), Wue, Wue, b2(bue),
      Wec, Wec, b2(bec), Wee, Wee, b2(bee),
      W_ut, W_em, Wv, b2(bv), Wo, b2(bo),
      jnp.broadcast_to(Wsp.reshape(1, EMB), (L, EMB)),
      jnp.broadcast_to(bsp.reshape(1, 1), (L, L)))
    s_ut = jnp.transpose(sut_k, (0, 2, 3, 1))
    s_em = jnp.transpose(sem_k, (0, 2, 3, 1))
    s_span = jnp.broadcast_to(span_k[:, :, :, None], (B, L, L, U - 1))
    return s_ut, s_em, s_span


# speedup vs baseline: 1.3839x; 1.3839x over previous
import jax
import jax.numpy as jnp
from jax.experimental import pallas as pl
from jax.experimental.pallas import tpu as pltpu
from jax.experimental.pallas import tpu_sc as plsc

B, L, U, F = 8, 64, 32, 2
VOCAB, EMB = 30522, 768
SPK_V, SPK_E = 16, 32
UT = 256
NEM = 7
BL = B * L

_SC_NC, _SC_NS = 2, 16
_SC_NW = _SC_NC * _SC_NS
_N_IDX = F * B * L
_ROWS_PER_W = _N_IDX // _SC_NW


def _sc_gather_body(table_hbm, idx_hbm, out_hbm, idx_v, rows_v, sem):
    wid = jax.lax.axis_index("s") * _SC_NC + jax.lax.axis_index("c")
    base = wid * _ROWS_PER_W
    pltpu.sync_copy(idx_hbm.at[pl.ds(base, _ROWS_PER_W)], idx_v)
    pltpu.async_copy(table_hbm.at[idx_v], rows_v, sem).wait()
    pltpu.sync_copy(rows_v, out_hbm.at[pl.ds(base, _ROWS_PER_W)])


def _sc_gather(table, idx):
    return pl.kernel(
        _sc_gather_body,
        out_type=jax.ShapeDtypeStruct((_N_IDX, EMB), jnp.float32),
        mesh=plsc.VectorSubcoreMesh(core_axis_name="c", subcore_axis_name="s"),
        scratch_types=[
            pltpu.VMEM((_ROWS_PER_W,), jnp.int32),
            pltpu.VMEM((_ROWS_PER_W, EMB), jnp.float32),
            pltpu.SemaphoreType.DMA,
        ],
    )(table, idx)


def _tc_body(rows_ref, spk_ref, pairm_ref, g_ref, spkt_ref,
             wucw_ref, wucs_ref, buc_ref, wuew_ref, wues_ref, bue_ref,
             wecw_ref, wecs_ref, bec_ref, weew_ref, wees_ref, bee_ref,
             wut_ref, wem_ref, wv_ref, bv_ref, wo_ref, bo_ref,
             wspr_ref, bspf_ref,
             sut_ref, sem_ref, sspan_ref):
    f32 = jnp.float32
    e0 = (rows_ref[0] + rows_ref[1]) * 0.5
    oh = (spk_ref[...]
          == jax.lax.broadcasted_iota(jnp.int32, (BL, SPK_V), 1)).astype(f32)
    spk = jnp.dot(oh, spkt_ref[...], preferred_element_type=f32)

    def ffn(ww, ws, bb):
        h = (jnp.dot(e0, ww[...], preferred_element_type=f32)
             + jnp.dot(spk, ws[...], preferred_element_type=f32)
             + bb[...])
        return jnp.where(h >= 0, h, 0.1 * h)

    utc = ffn(wucw_ref, wucs_ref, buc_ref)
    ute = ffn(wuew_ref, wues_ref, bue_ref)
    emc = ffn(wecw_ref, wecs_ref, bec_ref)
    eme = ffn(weew_ref, wees_ref, bee_ref)

    ones1 = jnp.ones((BL, 1), f32)

    def aug(x):
        return jnp.concatenate([x, ones1], axis=1)

    xc_ut, ye_ut = aug(utc), aug(ute)
    xc_em, ye_em = aug(emc), aug(eme)
    for o in range(2):
        xw = jnp.dot(xc_ut, wut_ref[o], preferred_element_type=f32)
        for b in range(B):
            sut_ref[b, o] = jax.lax.dot_general(
                xw[b * L:(b + 1) * L], ye_ut[b * L:(b + 1) * L],
                (((1,), (1,)), ((), ())), preferred_element_type=f32)
    for o in range(NEM):
        xw = jnp.dot(xc_em, wem_ref[o], preferred_element_type=f32)
        for b in range(B):
            sem_ref[b, o] = jax.lax.dot_general(
                xw[b * L:(b + 1) * L], ye_em[b * L:(b + 1) * L],
                (((1,), (1,)), ((), ())), preferred_element_type=f32)

    vp = jnp.dot(eme, wv_ref[...], preferred_element_type=f32) + bv_ref[...]
    sc = jnp.dot(vp, wo_ref[...], preferred_element_type=f32) + bo_ref[...]
    spw = jax.lax.dot_general(
        sc, wspr_ref[...], (((1,), (1,)), ((), ())),
        preferred_element_type=f32) + bspf_ref[...]
    spw = jnp.where(spw >= 0, spw, 0.1 * spw)
    spw3 = spw.reshape(B, L, L)
    m3 = (g_ref[...] != 0) & (pairm_ref[...] != 0)
    sm3 = jnp.where(m3, spw3, jnp.float32(-1.0))
    for b in range(B):
        sspan_ref[b] = jnp.broadcast_to(sm3[b][None], (U - 1, L, L))


def kernel(words, speakers, pad_mask, graphs, word_table, spk_table,
           Wuc, buc, Wue, bue, Wec, bec, Wee, bee, W_ut, W_em,
           Wq, bq, Wk, bk, Wv, bv, Wo, bo, Wsp, bsp):
    f32 = jnp.float32
    idx = jnp.transpose(words[:, :, 0, :], (2, 0, 1)).reshape(_N_IDX)
    idx = idx.astype(jnp.int32)
    rows = _sc_gather(word_table, idx)
    rows3 = rows.reshape(F, BL, EMB)

    spk_i = jnp.broadcast_to(
        speakers.reshape(BL)[:, None], (BL, SPK_V)).astype(jnp.int32)
    pair_i = (pad_mask[:, :, None] & pad_mask[:, None, :]).astype(jnp.int32)
    graphs_i = graphs.astype(jnp.int32)
    b2 = lambda v: v.reshape(1, -1).astype(f32)

    s_spec = pl.BlockSpec((SPK_E, UT), lambda i: (24, 0))
    in_specs = [
        pl.BlockSpec((F, BL, EMB), lambda i: (0, 0, 0)),
        pl.BlockSpec((BL, SPK_V), lambda i: (0, 0)),
        pl.BlockSpec((B, L, L), lambda i: (0, 0, 0)),
        pl.BlockSpec((B, L, L), lambda i: (0, 0, 0)),
        pl.BlockSpec((SPK_V, SPK_E), lambda i: (0, 0)),
        pl.BlockSpec((EMB, UT), lambda i: (0, 0)), s_spec, pl.BlockSpec((1, UT), lambda i: (0, 0)),
        pl.BlockSpec((EMB, UT), lambda i: (0, 0)), s_spec, pl.BlockSpec((1, UT), lambda i: (0, 0)),
        pl.BlockSpec((EMB, UT), lambda i: (0, 0)), s_spec, pl.BlockSpec((1, UT), lambda i: (0, 0)),
        pl.BlockSpec((EMB, UT), lambda i: (0, 0)), s_spec, pl.BlockSpec((1, UT), lambda i: (0, 0)),
        pl.BlockSpec((2, UT + 1, UT + 1), lambda i: (0, 0, 0)),
        pl.BlockSpec((NEM, UT + 1, UT + 1), lambda i: (0, 0, 0)),
        pl.BlockSpec((UT, EMB), lambda i: (0, 0)),
        pl.BlockSpec((1, EMB), lambda i: (0, 0)),
        pl.BlockSpec((EMB, EMB), lambda i: (0, 0)),
        pl.BlockSpec((1, EMB), lambda i: (0, 0)),
        pl.BlockSpec((L, EMB), lambda i: (0, 0)),
        pl.BlockSpec((1, L), lambda i: (0, 0)),
    ]
    out_specs = (
        pl.BlockSpec((B, 2, L, L), lambda i: (0, 0, 0, 0)),
        pl.BlockSpec((B, NEM, L, L), lambda i: (0, 0, 0, 0)),
        pl.BlockSpec((B, U - 1, L, L), lambda i: (0, 0, 0, 0)),
    )
    sut_k, sem_k, span_k = pl.pallas_call(
        _tc_body,
        grid=(1,),
        in_specs=in_specs,
        out_specs=out_specs,
        out_shape=(
            jax.ShapeDtypeStruct((B, 2, L, L), f32),
            jax.ShapeDtypeStruct((B, NEM, L, L), f32),
            jax.ShapeDtypeStruct((B, U - 1, L, L), f32),
        ),
    )(rows3, spk_i, pair_i, graphs_i, spk_table,
      Wuc, Wuc, b2(buc), Wue, Wue, b2(bue),
      Wec, Wec, b2(bec), Wee, Wee, b2(bee),
      W_ut, W_em, Wv, b2(bv), Wo, b2(bo),
      jnp.broadcast_to(Wsp.reshape(1, EMB), (L, EMB)),
      jnp.broadcast_to(bsp.reshape(1, 1), (1, L)))
    s_ut = jnp.transpose(sut_k, (0, 2, 3, 1))
    s_em = jnp.transpose(sem_k, (0, 2, 3, 1))
    s_span = jnp.transpose(span_k, (0, 2, 3, 1))
    return s_ut, s_em, s_span
